# baseline (speedup 1.0000x reference)
import numpy as np
import jax
import jax.numpy as jnp
from jax.experimental import pallas as pl
from jax.experimental.pallas import tpu as pltpu

_B = 128
_V = 100000
_C = 2048
_NB = (_V + _C - 1) // _C

_TINY = np.float32(np.finfo(np.float32).tiny)


def _sampler_body(x_ref, out_ref, bestv_ref, besti_ref):
    j = pl.program_id(0)

    @pl.when(j == 0)
    def _init():
        bestv_ref[...] = jnp.full((_B, 1), -jnp.inf, jnp.float32)
        besti_ref[...] = jnp.zeros((_B, 1), jnp.int32)

    x = x_ref[...]
    col = jax.lax.broadcasted_iota(jnp.int32, (_B, _C), 1) + j * _C
    row = jax.lax.broadcasted_iota(jnp.int32, (_B, _C), 0)
    cnt = (row * _V + col).astype(jnp.uint32)

    ks = (jnp.uint32(0), jnp.uint32(1), jnp.uint32(0x1BD11BDB))
    rot = ((13, 15, 26, 6), (17, 29, 16, 24))
    x0 = jnp.zeros((_B, _C), jnp.uint32) + ks[0]
    x1 = cnt + ks[1]
    for r in range(5):
        for rr in rot[r % 2]:
            x0 = x0 + x1
            x1 = (x1 << jnp.uint32(rr)) | (x1 >> jnp.uint32(32 - rr))
            x1 = x0 ^ x1
        x0 = x0 + ks[(r + 1) % 3]
        x1 = x1 + ks[(r + 2) % 3] + jnp.uint32(r + 1)
    bits = x0 ^ x1

    fb = (bits >> jnp.uint32(9)) | jnp.uint32(0x3F800000)
    f = jax.lax.bitcast_convert_type(fb, jnp.float32) - jnp.float32(1.0)
    u = jnp.maximum(f, _TINY)
    g = -jnp.log(-jnp.log(u))

    v = x + g
    v = jnp.where(col < _V, v, -jnp.inf)

    m = jnp.max(v, axis=1, keepdims=True)
    cand = jnp.where(v == m, col, jnp.int32(0x7FFFFFFF))
    idx = jnp.min(cand, axis=1, keepdims=True)

    upd = m > bestv_ref[...]
    bestv_ref[...] = jnp.where(upd, m, bestv_ref[...])
    besti_ref[...] = jnp.where(upd, idx, besti_ref[...])

    @pl.when(j == _NB - 1)
    def _fin():
        out_ref[...] = besti_ref[...]


def kernel(logits):
    out = pl.pallas_call(
        _sampler_body,
        grid=(_NB,),
        in_specs=[pl.BlockSpec((_B, _C), lambda j: (0, j))],
        out_specs=pl.BlockSpec((_B, 1), lambda j: (0, 0)),
        out_shape=jax.ShapeDtypeStruct((_B, 1), jnp.int32),
        scratch_shapes=[
            pltpu.VMEM((_B, 1), jnp.float32),
            pltpu.VMEM((_B, 1), jnp.int32),
        ],
    )(logits)
    return out.reshape(_B)
